# Initial kernel scaffold; baseline (speedup 1.0000x reference)
import jax
import jax.numpy as jnp
from jax.experimental import pallas as pl


def kernel(grad_last, input):
    raise NotImplementedError("write your pallas kernel here")


# speedup vs baseline: 8.6351x; 8.6351x over previous
import jax
import jax.numpy as jnp
from jax.experimental import pallas as pl

_START, _END = 512, 2560
_BLOCK = 512


def _pad_kernel(g_ref, o_ref):
    j = pl.program_id(1)
    lo = _START // _BLOCK
    hi = _END // _BLOCK
    inside = (j >= lo) & (j < hi)

    @pl.when(inside)
    def _():
        o_ref[...] = g_ref[...]

    @pl.when(jnp.logical_not(inside))
    def _():
        o_ref[...] = jnp.zeros_like(o_ref)


def kernel(grad_last, input):
    b0, b1, g_rows, feat = grad_last.shape
    seq = input.shape[1]
    nb = b0 * b1
    g = grad_last.reshape(nb, g_rows, feat)
    lo = _START // _BLOCK
    n_g_blocks = g_rows // _BLOCK

    out = pl.pallas_call(
        _pad_kernel,
        grid=(nb, seq // _BLOCK),
        in_specs=[
            pl.BlockSpec(
                (1, _BLOCK, feat),
                lambda b, j: (b, jnp.clip(j - lo, 0, n_g_blocks - 1), 0),
            )
        ],
        out_specs=pl.BlockSpec((1, _BLOCK, feat), lambda b, j: (b, j, 0)),
        out_shape=jax.ShapeDtypeStruct((nb, seq, feat), grad_last.dtype),
    )(g)
    return out.reshape(b0, b1, seq, feat)


# speedup vs baseline: 8.6784x; 1.0050x over previous
import jax
import jax.numpy as jnp
from jax.experimental import pallas as pl
from jax.experimental.pallas import tpu as pltpu

_START, _END = 512, 2560
_BLOCK = 512


def _pad_kernel(g_ref, o_ref):
    j = pl.program_id(1)
    lo = _START // _BLOCK
    hi = _END // _BLOCK
    inside = (j >= lo) & (j < hi)

    @pl.when(inside)
    def _():
        o_ref[...] = g_ref[...]

    @pl.when(jnp.logical_not(inside))
    def _():
        o_ref[...] = jnp.zeros_like(o_ref)


def kernel(grad_last, input):
    b0, b1, g_rows, feat = grad_last.shape
    seq = input.shape[1]
    nb = b0 * b1
    g = grad_last.reshape(nb, g_rows, feat)
    lo = _START // _BLOCK
    n_g_blocks = g_rows // _BLOCK

    out = pl.pallas_call(
        _pad_kernel,
        grid=(nb, seq // _BLOCK),
        in_specs=[
            pl.BlockSpec(
                (1, _BLOCK, feat),
                lambda b, j: (b, jnp.clip(j - lo, 0, n_g_blocks - 1), 0),
            )
        ],
        out_specs=pl.BlockSpec((1, _BLOCK, feat), lambda b, j: (b, j, 0)),
        out_shape=jax.ShapeDtypeStruct((nb, seq, feat), grad_last.dtype),
        compiler_params=pltpu.CompilerParams(
            dimension_semantics=("parallel", "parallel"),
        ),
    )(g)
    return out.reshape(b0, b1, seq, feat)


# speedup vs baseline: 10.9794x; 1.2651x over previous
import jax
import jax.numpy as jnp
from jax.experimental import pallas as pl
from jax.experimental.pallas import tpu as pltpu

_START, _END = 512, 2560
_BLOCK = 512


def _pad_kernel(g_ref, o_ref):
    j = pl.program_id(1)
    lo = _START // _BLOCK
    hi = _END // _BLOCK
    inside = (j >= lo) & (j < hi)

    @pl.when(inside)
    def _():
        o_ref[...] = g_ref[...]

    @pl.when(jnp.logical_not(inside))
    def _():
        o_ref[...] = jnp.zeros_like(o_ref)


def kernel(grad_last, input):
    b0, b1, g_rows, feat = grad_last.shape
    seq = input.shape[1]
    nb = b0 * b1
    g = grad_last.reshape(nb, g_rows, feat)
    lo = _START // _BLOCK
    n_g_blocks = g_rows // _BLOCK

    out = pl.pallas_call(
        _pad_kernel,
        grid=(1, seq // _BLOCK),
        in_specs=[
            pl.BlockSpec(
                (nb, _BLOCK, feat),
                lambda b, j: (b, jnp.clip(j - lo, 0, n_g_blocks - 1), 0),
            )
        ],
        out_specs=pl.BlockSpec((nb, _BLOCK, feat), lambda b, j: (b, j, 0)),
        out_shape=jax.ShapeDtypeStruct((nb, seq, feat), grad_last.dtype),
        compiler_params=pltpu.CompilerParams(
            dimension_semantics=("parallel", "parallel"),
        ),
    )(g)
    return out.reshape(b0, b1, seq, feat)
